# baseline (speedup 1.0000x reference)
import functools

import jax
import jax.numpy as jnp
from jax import lax
from jax.experimental import pallas as pl
from jax.experimental.pallas import tpu as pltpu
from jax.experimental.pallas import tpu_sc as plsc

B = 16384
V = 1000
VP = 1024
D = 128
L = 16
NBLK = VP // D

_info = plsc.get_sparse_core_info()
NC = _info.num_cores
NS = _info.num_subcores
NW = NC * NS
BPW = B // NW
CH = 128
NCHUNK = BPW // CH

_EPS = 1e-12


def _tc_body(table_ref, g2_ref):
    t = table_ref[...]
    n = t * lax.rsqrt(jnp.maximum(jnp.sum(t * t, axis=1, keepdims=True), _EPS))
    npad = jnp.concatenate([n, jnp.zeros((VP - V, D), jnp.float32)], axis=0)
    for k in range(NBLK):
        rk = npad[k * D:(k + 1) * D, :]
        g2_ref[pl.ds(k * VP, V), :] = lax.dot_general(
            n, rk, (((1,), (1,)), ((), ())), preferred_element_type=jnp.float32
        )


_tc_kernel = pl.pallas_call(
    _tc_body,
    out_shape=jax.ShapeDtypeStruct((NBLK * VP, D), jnp.float32),
)


def _sc_body(center_hbm, context_hbm, gram_hbm, out_hbm, cidx, xidx, fidx, sims, sem):
    wid = lax.axis_index("s") * NC + lax.axis_index("c")
    base = wid * BPW

    for j in range(NCHUNK):
        pltpu.sync_copy(center_hbm.at[pl.ds(base + j * CH, CH)], cidx.at[j])
        pltpu.sync_copy(context_hbm.at[pl.ds(base + j * CH, CH)], xidx.at[j])

    copies = []
    for j in range(NCHUNK):
        for g in range(CH // L):
            c = cidx[j, pl.ds(g * L, L)]
            x = xidx[j, pl.ds(g * L, L)]
            f = (
                lax.shift_left(lax.shift_right_logical(x, 7), 17)
                + lax.shift_left(c, 7)
                + (x & 127)
            )
            fidx[j, pl.ds(g * L, L)] = f
        copies.append(
            pltpu.async_copy(gram_hbm.at[fidx.at[j]], sims.at[pl.ds(j * CH, CH)], sem)
        )
    for c in copies:
        c.wait()

    pltpu.sync_copy(sims, out_hbm.at[pl.ds(base, BPW)])


_mesh = plsc.VectorSubcoreMesh(core_axis_name="c", subcore_axis_name="s")

_sc_kernel = functools.partial(
    pl.kernel,
    out_type=jax.ShapeDtypeStruct((B,), jnp.float32),
    mesh=_mesh,
    scratch_types=[
        pltpu.VMEM((NCHUNK, CH), jnp.int32),
        pltpu.VMEM((NCHUNK, CH), jnp.int32),
        pltpu.VMEM((NCHUNK, CH), jnp.int32),
        pltpu.VMEM((BPW,), jnp.float32),
        pltpu.SemaphoreType.DMA,
    ],
    compiler_params=pltpu.CompilerParams(
        skip_device_barrier=True,
        disable_bounds_checks=True,
    ),
)(_sc_body)


def kernel(center, context, embedding_table):
    gram2 = _tc_kernel(embedding_table)
    sims = _sc_kernel(center, context, gram2.reshape(NBLK * VP * D))
    return sims.reshape(B, 1)


# speedup vs baseline: 1.1372x; 1.1372x over previous
import functools

import jax
import jax.numpy as jnp
from jax import lax
from jax.experimental import pallas as pl
from jax.experimental.pallas import tpu as pltpu
from jax.experimental.pallas import tpu_sc as plsc

B = 16384
V = 1000
VP = 1024
D = 128
L = 16
NBLK = VP // D

_info = plsc.get_sparse_core_info()
NC = _info.num_cores
NS = _info.num_subcores
NW = NC * NS
BPW = B // NW
CH = 128
NCHUNK = BPW // CH

_EPS = 1e-12


def _tc_body(table_ref, g2_ref):
    t = table_ref[...]
    n = t * lax.rsqrt(jnp.maximum(jnp.sum(t * t, axis=1, keepdims=True), _EPS))
    npad = jnp.concatenate([n, jnp.zeros((VP - V, D), jnp.float32)], axis=0)
    for k in range(NBLK):
        rk = npad[k * D:(k + 1) * D, :]
        g2_ref[pl.ds(k * VP, V), :] = lax.dot_general(
            n, rk, (((1,), (1,)), ((), ())), preferred_element_type=jnp.float32
        )


_tc_kernel = pl.pallas_call(
    _tc_body,
    out_shape=jax.ShapeDtypeStruct((NBLK * VP, D), jnp.float32),
)


def _sc_body(center_hbm, context_hbm, gram_hbm, out_hbm, cidx, xidx, fidx, sims, sem,
             sem2):
    wid = lax.axis_index("s") * NC + lax.axis_index("c")
    base = wid * BPW

    stage = []
    for j in range(NCHUNK):
        stage.append(
            pltpu.async_copy(center_hbm.at[pl.ds(base + j * CH, CH)], cidx.at[j], sem2)
        )
        stage.append(
            pltpu.async_copy(context_hbm.at[pl.ds(base + j * CH, CH)], xidx.at[j], sem2)
        )
    for s in stage:
        s.wait()

    copies = []
    for j in range(NCHUNK):
        for g in range(CH // L):
            c = cidx[j, pl.ds(g * L, L)]
            x = xidx[j, pl.ds(g * L, L)]
            f = (
                lax.shift_left(lax.shift_right_logical(x, 7), 17)
                + lax.shift_left(c, 7)
                + (x & 127)
            )
            fidx[j, pl.ds(g * L, L)] = f
        copies.append(
            pltpu.async_copy(gram_hbm.at[fidx.at[j]], sims.at[pl.ds(j * CH, CH)], sem)
        )
    for c in copies:
        c.wait()

    pltpu.sync_copy(sims, out_hbm.at[pl.ds(base, BPW)])


_mesh = plsc.VectorSubcoreMesh(core_axis_name="c", subcore_axis_name="s")

_sc_kernel = functools.partial(
    pl.kernel,
    out_type=jax.ShapeDtypeStruct((B,), jnp.float32),
    mesh=_mesh,
    scratch_types=[
        pltpu.VMEM((NCHUNK, CH), jnp.int32),
        pltpu.VMEM((NCHUNK, CH), jnp.int32),
        pltpu.VMEM((NCHUNK, CH), jnp.int32),
        pltpu.VMEM((BPW,), jnp.float32),
        pltpu.SemaphoreType.DMA,
        pltpu.SemaphoreType.DMA,
    ],
    compiler_params=pltpu.CompilerParams(
        skip_device_barrier=True,
        disable_bounds_checks=True,
    ),
)(_sc_body)


def kernel(center, context, embedding_table):
    gram2 = _tc_kernel(embedding_table)
    sims = _sc_kernel(center, context, gram2.reshape(NBLK * VP * D))
    return sims.reshape(B, 1)
